# baseline (speedup 1.0000x reference)
import jax
import jax.numpy as jnp
from jax.experimental import pallas as pl
from jax.experimental.pallas import tpu as pltpu

_ROWS = 8


def _kern(actions_ref, logits_ref, lp_ref, mode_ref):
    x = logits_ref[...]
    col = jax.lax.broadcasted_iota(jnp.int32, x.shape, 1)

    m = jnp.max(x, axis=1, keepdims=True)
    s = jnp.sum(jnp.exp(x - m), axis=1, keepdims=True)

    big = jnp.int32(2 ** 30)
    amax = jnp.min(jnp.where(x == m, col, big), axis=1, keepdims=True)

    a = actions_ref[...]
    g = jnp.sum(jnp.where(col == a, x, 0.0), axis=1, keepdims=True)

    lp_ref[...] = g - (jnp.log(s) + m)
    mode_ref[...] = amax


@jax.jit
def kernel(logits, actions):
    b, n = logits.shape
    nb = b // _ROWS
    actions = actions.astype(jnp.int32)
    lp, mode = pl.pallas_call(
        _kern,
        grid=(nb,),
        in_specs=[
            pl.BlockSpec((_ROWS, 1), lambda i: (i, 0)),
            pl.BlockSpec((_ROWS, n), lambda i: (i, 0)),
        ],
        out_specs=[
            pl.BlockSpec((_ROWS, 1), lambda i: (i, 0)),
            pl.BlockSpec((_ROWS, 1), lambda i: (i, 0)),
        ],
        out_shape=[
            jax.ShapeDtypeStruct((b, 1), jnp.float32),
            jax.ShapeDtypeStruct((b, 1), jnp.int32),
        ],
        compiler_params=pltpu.CompilerParams(
            dimension_semantics=("parallel",),
        ),
    )(actions, logits)
    return lp, mode


# speedup vs baseline: 1.2085x; 1.2085x over previous
import jax
import jax.numpy as jnp
from jax.experimental import pallas as pl
from jax.experimental.pallas import tpu as pltpu

_ROWS = 8
_W = 2048


def _kern(actions_ref, logits_ref, lp_ref, mode_ref):
    n = logits_ref.shape[1]
    nfull = n // _W
    tail_base = nfull * _W
    over_base = n - _W

    m_acc = logits_ref[:, pl.ds(0, _W)]
    for c in range(1, nfull):
        m_acc = jnp.maximum(m_acc, logits_ref[:, pl.ds(c * _W, _W)])
    m_acc = jnp.maximum(m_acc, logits_ref[:, pl.ds(over_base, _W)])
    m = jnp.max(m_acc, axis=1, keepdims=True)

    l2e = jnp.float32(1.4426950408889634)
    m2 = m * l2e
    big = jnp.int32(2 ** 30)
    iota = jax.lax.broadcasted_iota(jnp.int32, (_ROWS, _W), 1)
    s_acc = jnp.zeros((_ROWS, _W), jnp.float32)
    i_acc = jnp.full((_ROWS, _W), big, jnp.int32)
    for c in range(nfull):
        x = logits_ref[:, pl.ds(c * _W, _W)]
        s_acc = s_acc + jnp.exp2(x * l2e - m2)
        col = iota + jnp.int32(c * _W)
        i_acc = jnp.minimum(i_acc, jnp.where(x == m, col, big))
    x = logits_ref[:, pl.ds(over_base, _W)]
    col = iota + jnp.int32(over_base)
    s_acc = s_acc + jnp.where(col >= tail_base,
                              jnp.exp2(x * l2e - m2), 0.0)
    i_acc = jnp.minimum(i_acc, jnp.where(x == m, col, big))

    s = jnp.sum(s_acc, axis=1, keepdims=True)
    amax = jnp.min(i_acc, axis=1, keepdims=True)

    lane = jax.lax.broadcasted_iota(jnp.int32, (1, 128), 1)
    vals = []
    for i in range(_ROWS):
        a = actions_ref[i, 0]
        base = pl.multiple_of((a // 128) * 128, 128)
        v = logits_ref[pl.ds(i, 1), pl.ds(base, 128)]
        vals.append(jnp.sum(jnp.where(lane == a % 128, v, 0.0),
                            axis=1, keepdims=True))
    g = jnp.concatenate(vals, axis=0)

    lp_ref[...] = g - (jnp.log(s) + m)
    mode_ref[...] = amax


@jax.jit
def kernel(logits, actions):
    b, n = logits.shape
    nb = b // _ROWS
    actions = actions.astype(jnp.int32)
    lp, mode = pl.pallas_call(
        _kern,
        grid=(nb,),
        in_specs=[
            pl.BlockSpec((_ROWS, 1), lambda i: (i, 0), memory_space=pltpu.SMEM),
            pl.BlockSpec((_ROWS, n), lambda i: (i, 0)),
        ],
        out_specs=[
            pl.BlockSpec((_ROWS, 1), lambda i: (i, 0)),
            pl.BlockSpec((_ROWS, 1), lambda i: (i, 0)),
        ],
        out_shape=[
            jax.ShapeDtypeStruct((b, 1), jnp.float32),
            jax.ShapeDtypeStruct((b, 1), jnp.int32),
        ],
        compiler_params=pltpu.CompilerParams(
            dimension_semantics=("parallel",),
        ),
    )(actions, logits)
    return lp, mode


# speedup vs baseline: 2.8362x; 2.3468x over previous
import jax
import jax.numpy as jnp
from jax.experimental import pallas as pl
from jax.experimental.pallas import tpu as pltpu

_VB = 5000
_C = 40
_L2E = 1.4426950408889634
_LN2 = 0.6931471805599453
_BIG = 2 ** 30


def _kern(a_ref, lt_ref, lp_ref, md_ref, m_ref, i_ref, s_ref, g_ref):
    j = pl.program_id(0)
    nb = pl.num_programs(0)
    sub = jax.lax.broadcasted_iota(jnp.int32, (_C, 128), 0)
    a = a_ref[...]

    @pl.when(j == 0)
    def _init():
        m_ref[...] = jnp.full_like(m_ref, -jnp.inf)
        i_ref[...] = jnp.zeros_like(i_ref)
        s_ref[...] = jnp.zeros_like(s_ref)
        g_ref[...] = jnp.zeros_like(g_ref)

    m_old = m_ref[...]
    m_acc = m_old
    i_acc = i_ref[...]
    g_acc = g_ref[...]
    base = j * _VB

    for c in range(_VB // _C):
        o = base + c * _C
        x = lt_ref[pl.ds(c * _C, _C), :]
        gt = x > m_acc
        i_acc = jnp.where(gt, jnp.int32(o), i_acc)
        m_acc = jnp.maximum(x, m_acc)
        g_acc = g_acc + jnp.where(sub == (a - o), x, 0.0)

    s_acc = s_ref[...] * jnp.exp2((m_old - m_acc) * _L2E)
    m2 = m_acc * _L2E

    for c in range(_VB // _C):
        x = lt_ref[pl.ds(c * _C, _C), :]
        s_acc = s_acc + jnp.exp2(x * _L2E - m2)

    m_ref[...] = m_acc
    i_ref[...] = i_acc
    s_ref[...] = s_acc
    g_ref[...] = g_acc

    @pl.when(j == nb - 1)
    def _fin():
        m_f = jnp.max(m_acc, axis=0, keepdims=True)
        vi = i_acc + sub
        i_f = jnp.min(jnp.where(m_acc == m_f, vi, _BIG), axis=0,
                      keepdims=True)
        s_f = jnp.sum(s_acc * jnp.exp2((m_acc - m_f) * _L2E), axis=0,
                      keepdims=True)
        g_f = jnp.sum(g_acc, axis=0, keepdims=True)
        lp_ref[...] = g_f - (m_f + _LN2 * jnp.log2(s_f))
        md_ref[...] = i_f


@jax.jit
def kernel(logits, actions):
    b, n = logits.shape
    lt = logits.T
    av = actions.reshape(1, b).astype(jnp.int32)
    nb = n // _VB
    lp, mode = pl.pallas_call(
        _kern,
        grid=(nb,),
        in_specs=[
            pl.BlockSpec((1, b), lambda j: (0, 0)),
            pl.BlockSpec((_VB, b), lambda j: (j, 0)),
        ],
        out_specs=[
            pl.BlockSpec((1, b), lambda j: (0, 0)),
            pl.BlockSpec((1, b), lambda j: (0, 0)),
        ],
        out_shape=[
            jax.ShapeDtypeStruct((1, b), jnp.float32),
            jax.ShapeDtypeStruct((1, b), jnp.int32),
        ],
        scratch_shapes=[
            pltpu.VMEM((_C, b), jnp.float32),
            pltpu.VMEM((_C, b), jnp.int32),
            pltpu.VMEM((_C, b), jnp.float32),
            pltpu.VMEM((_C, b), jnp.float32),
        ],
    )(av, lt)
    return lp.reshape(b, 1), mode.reshape(b, 1)


# speedup vs baseline: 2.9249x; 1.0313x over previous
import jax
import jax.numpy as jnp
from jax.experimental import pallas as pl
from jax.experimental.pallas import tpu as pltpu

_VB = 10000
_C = 40
_L2E = 1.4426950408889634
_LN2 = 0.6931471805599453
_BIG = 2 ** 30


def _kern(a_ref, lt_ref, lp_ref, md_ref, m_ref, i_ref, s_ref, g_ref):
    j = pl.program_id(0)
    nb = pl.num_programs(0)
    sub = jax.lax.broadcasted_iota(jnp.int32, (_C, 128), 0)
    a = a_ref[...]

    @pl.when(j == 0)
    def _init():
        m_ref[...] = jnp.full_like(m_ref, -jnp.inf)
        i_ref[...] = jnp.zeros_like(i_ref)
        s_ref[...] = jnp.zeros_like(s_ref)
        g_ref[...] = jnp.zeros_like(g_ref)

    m_old = m_ref[...]
    m_acc = m_old
    i_acc = i_ref[...]
    g_acc = g_ref[...]
    base = j * _VB

    for c in range(_VB // _C):
        o = base + c * _C
        x = lt_ref[pl.ds(c * _C, _C), :]
        gt = x > m_acc
        i_acc = jnp.where(gt, jnp.int32(o), i_acc)
        m_acc = jnp.maximum(x, m_acc)
        g_acc = g_acc + jnp.where(sub == (a - o), x, 0.0)

    s_acc = s_ref[...] * jnp.exp2((m_old - m_acc) * _L2E)
    m2 = m_acc * _L2E

    for c in range(_VB // _C):
        x = lt_ref[pl.ds(c * _C, _C), :]
        s_acc = s_acc + jnp.exp2(x * _L2E - m2)

    m_ref[...] = m_acc
    i_ref[...] = i_acc
    s_ref[...] = s_acc
    g_ref[...] = g_acc

    @pl.when(j == nb - 1)
    def _fin():
        m_f = jnp.max(m_acc, axis=0, keepdims=True)
        vi = i_acc + sub
        i_f = jnp.min(jnp.where(m_acc == m_f, vi, _BIG), axis=0,
                      keepdims=True)
        s_f = jnp.sum(s_acc * jnp.exp2((m_acc - m_f) * _L2E), axis=0,
                      keepdims=True)
        g_f = jnp.sum(g_acc, axis=0, keepdims=True)
        lp_ref[...] = g_f - (m_f + _LN2 * jnp.log2(s_f))
        md_ref[...] = i_f


@jax.jit
def kernel(logits, actions):
    b, n = logits.shape
    lt = logits.T
    av = actions.reshape(1, b).astype(jnp.int32)
    nb = n // _VB
    lp, mode = pl.pallas_call(
        _kern,
        grid=(nb,),
        in_specs=[
            pl.BlockSpec((1, b), lambda j: (0, 0)),
            pl.BlockSpec((_VB, b), lambda j: (j, 0)),
        ],
        out_specs=[
            pl.BlockSpec((1, b), lambda j: (0, 0)),
            pl.BlockSpec((1, b), lambda j: (0, 0)),
        ],
        out_shape=[
            jax.ShapeDtypeStruct((1, b), jnp.float32),
            jax.ShapeDtypeStruct((1, b), jnp.int32),
        ],
        scratch_shapes=[
            pltpu.VMEM((_C, b), jnp.float32),
            pltpu.VMEM((_C, b), jnp.int32),
            pltpu.VMEM((_C, b), jnp.float32),
            pltpu.VMEM((_C, b), jnp.float32),
        ],
    )(av, lt)
    return lp.reshape(b, 1), mode.reshape(b, 1)
